# Initial kernel scaffold; baseline (speedup 1.0000x reference)
import jax
import jax.numpy as jnp
from jax.experimental import pallas as pl


def kernel(x, data):
    raise NotImplementedError("write your pallas kernel here")


# speedup vs baseline: 1.0230x; 1.0230x over previous
import functools

import jax
import jax.numpy as jnp
from jax import lax
from jax.experimental import pallas as pl
from jax.experimental.pallas import tpu as pltpu
from jax.experimental.pallas import tpu_sc as plsc

D = 32
CHUNK = 128
NC = 2
NS = 16
NW = NC * NS


@functools.partial(jax.jit, static_argnums=())
def _sc_gather(x2d, data):
    n_chunks, _ = x2d.shape
    chunks_per_w = n_chunks // NW
    total_rows = n_chunks * CHUNK

    mesh = plsc.VectorSubcoreMesh(core_axis_name="c", subcore_axis_name="s")

    @functools.partial(
        pl.kernel,
        out_type=jax.ShapeDtypeStruct((total_rows, D), jnp.float32),
        mesh=mesh,
        scratch_types=[
            pltpu.VMEM((chunks_per_w, CHUNK), jnp.int32),
            pltpu.VMEM((CHUNK, D), jnp.float32),
            pltpu.SemaphoreType.DMA,
        ],
        compiler_params=pltpu.CompilerParams(use_tc_tiling_on_sc=False),
    )
    def k(x_hbm, data_hbm, out_hbm, idx_v, rows_v, gsem):
        wid = lax.axis_index("s") * NC + lax.axis_index("c")
        base = wid * chunks_per_w
        pltpu.sync_copy(x_hbm.at[pl.ds(base, chunks_per_w)], idx_v)

        def body(j, carry):
            pltpu.async_copy(data_hbm.at[idx_v.at[j]], rows_v, gsem).wait()
            pltpu.sync_copy(rows_v, out_hbm.at[pl.ds((base + j) * CHUNK, CHUNK)])
            return carry

        lax.fori_loop(0, chunks_per_w, body, 0)

    return k(x2d, data)


def kernel(x, data):
    xf = x.reshape(-1, CHUNK)
    out = _sc_gather(xf, data)
    return out.reshape(x.shape + (D,))


# speedup vs baseline: 1.1026x; 1.0778x over previous
import functools

import jax
import jax.numpy as jnp
from jax import lax
from jax.experimental import pallas as pl
from jax.experimental.pallas import tpu as pltpu
from jax.experimental.pallas import tpu_sc as plsc

D = 32
CHUNK = 1024
NC = 2
NS = 16
NW = NC * NS


@functools.partial(jax.jit, static_argnums=())
def _sc_gather(x2d, data):
    n_chunks, _ = x2d.shape
    chunks_per_w = n_chunks // NW
    total_rows = n_chunks * CHUNK

    mesh = plsc.VectorSubcoreMesh(core_axis_name="c", subcore_axis_name="s")

    @functools.partial(
        pl.kernel,
        out_type=jax.ShapeDtypeStruct((total_rows, D), jnp.float32),
        mesh=mesh,
        scratch_types=[
            pltpu.VMEM((chunks_per_w, CHUNK), jnp.int32),
            pltpu.VMEM((CHUNK, D), jnp.float32),
            pltpu.SemaphoreType.DMA,
        ],
        compiler_params=pltpu.CompilerParams(use_tc_tiling_on_sc=False),
    )
    def k(x_hbm, data_hbm, out_hbm, idx_v, rows_v, gsem):
        wid = lax.axis_index("s") * NC + lax.axis_index("c")
        base = wid * chunks_per_w
        pltpu.sync_copy(x_hbm.at[pl.ds(base, chunks_per_w)], idx_v)

        def body(j, carry):
            pltpu.async_copy(data_hbm.at[idx_v.at[j]], rows_v, gsem).wait()
            pltpu.sync_copy(rows_v, out_hbm.at[pl.ds((base + j) * CHUNK, CHUNK)])
            return carry

        lax.fori_loop(0, chunks_per_w, body, 0)

    return k(x2d, data)


def kernel(x, data):
    xf = x.reshape(-1, CHUNK)
    out = _sc_gather(xf, data)
    return out.reshape(x.shape + (D,))


# speedup vs baseline: 1.7900x; 1.6234x over previous
import functools

import jax
import jax.numpy as jnp
from jax import lax
from jax.experimental import pallas as pl
from jax.experimental.pallas import tpu as pltpu
from jax.experimental.pallas import tpu_sc as plsc

B = 16384
H = 50
D = 32
NC = 2
NS = 16
NW = NC * NS
RB = B // NW
CB = 32
NCH = RB // CB


def _sc_gather(x, data):
    mesh = plsc.VectorSubcoreMesh(core_axis_name="c", subcore_axis_name="s")

    @functools.partial(
        pl.kernel,
        out_type=jax.ShapeDtypeStruct((B, H, D), jnp.float32),
        mesh=mesh,
        scratch_types=[
            pltpu.VMEM((RB, H), jnp.int32),
            pltpu.VMEM((CB, H, D), jnp.float32),
            pltpu.SemaphoreType.DMA,
        ],
        compiler_params=pltpu.CompilerParams(use_tc_tiling_on_sc=False),
    )
    def k(x_hbm, data_hbm, out_hbm, idx_v, rows_v, gsem):
        wid = lax.axis_index("s") * NC + lax.axis_index("c")
        base = wid * RB
        pltpu.sync_copy(x_hbm.at[pl.ds(base, RB)], idx_v)

        def body(g, carry):
            def fire(j, c):
                pltpu.async_copy(
                    data_hbm.at[idx_v.at[g * CB + j]], rows_v.at[j], gsem
                )
                return c

            lax.fori_loop(0, CB, fire, 0)
            pltpu.make_async_copy(
                out_hbm.at[pl.ds(base + g * CB, CB)], rows_v, gsem
            ).wait()
            pltpu.sync_copy(rows_v, out_hbm.at[pl.ds(base + g * CB, CB)])
            return carry

        lax.fori_loop(0, NCH, body, 0)

    return k(x, data)


def kernel(x, data):
    return _sc_gather(x, data)


# speedup vs baseline: 2.1976x; 1.2277x over previous
import functools

import jax
import jax.numpy as jnp
from jax import lax
from jax.experimental import pallas as pl
from jax.experimental.pallas import tpu as pltpu
from jax.experimental.pallas import tpu_sc as plsc

B = 16384
H = 50
D = 32
NC = 2
NS = 16
NW = NC * NS
RB = B // NW
TPAD = RB + 8


def _sc_gather(xT, data):
    mesh = plsc.VectorSubcoreMesh(core_axis_name="c", subcore_axis_name="s")

    @functools.partial(
        pl.kernel,
        out_type=jax.ShapeDtypeStruct((H, D, B), jnp.float32),
        mesh=mesh,
        scratch_types=[
            pltpu.VMEM((H, RB), jnp.int32),
            pltpu.VMEM((RB, D), jnp.float32),
            pltpu.VMEM((D, TPAD), jnp.float32),
            pltpu.SemaphoreType.DMA,
        ],
        compiler_params=pltpu.CompilerParams(
            use_tc_tiling_on_sc=False, needs_layout_passes=False
        ),
    )
    def k(xT_hbm, data_hbm, outT_hbm, idxT_v, rows_v, trans_v, gsem):
        wid = lax.axis_index("s") * NC + lax.axis_index("c")
        b0 = wid * RB
        pltpu.sync_copy(xT_hbm.at[:, pl.ds(b0, RB)], idxT_v)
        lanes = lax.iota(jnp.int32, 16)

        def h_body(h, carry):
            pltpu.async_copy(data_hbm.at[idxT_v.at[h]], rows_v, gsem).wait()

            def b_body(b, c2):
                v0 = rows_v[b, pl.ds(0, 16)]
                v1 = rows_v[b, pl.ds(16, 16)]
                bb = jnp.full((16,), 0, jnp.int32) + b
                plsc.store_scatter(trans_v, [lanes, bb], v0)
                plsc.store_scatter(trans_v, [lanes + 16, bb], v1)
                return c2

            lax.fori_loop(0, RB, b_body, 0)
            pltpu.sync_copy(
                trans_v.at[:, pl.ds(0, RB)], outT_hbm.at[h, :, pl.ds(b0, RB)]
            )
            return carry

        lax.fori_loop(0, H, h_body, 0)

    return k(xT, data)


def kernel(x, data):
    outT = _sc_gather(x.T, data)
    return jnp.transpose(outT, (2, 0, 1))


# speedup vs baseline: 2.4428x; 1.1116x over previous
import functools

import jax
import jax.numpy as jnp
from jax import lax
from jax.experimental import pallas as pl
from jax.experimental.pallas import tpu as pltpu
from jax.experimental.pallas import tpu_sc as plsc

B = 16384
H = 50
D = 32
NC = 2
NS = 16
NW = NC * NS
RB = B // NW
TPAD = RB + 8
UNROLL = 8


def _sc_gather(xT, data):
    mesh = plsc.VectorSubcoreMesh(core_axis_name="c", subcore_axis_name="s")

    @functools.partial(
        pl.kernel,
        out_type=jax.ShapeDtypeStruct((H, D, B), jnp.float32),
        mesh=mesh,
        scratch_types=[
            pltpu.VMEM((H, RB), jnp.int32),
            pltpu.VMEM((2, RB, D), jnp.float32),
            pltpu.VMEM((D, TPAD), jnp.float32),
            pltpu.SemaphoreType.DMA,
            pltpu.SemaphoreType.DMA,
        ],
        compiler_params=pltpu.CompilerParams(
            use_tc_tiling_on_sc=False, needs_layout_passes=False
        ),
    )
    def k(xT_hbm, data_hbm, outT_hbm, idxT_v, rows_v, trans_v, gsem0, gsem1):
        wid = lax.axis_index("s") * NC + lax.axis_index("c")
        b0 = wid * RB
        pltpu.sync_copy(xT_hbm.at[:, pl.ds(b0, RB)], idxT_v)
        lanes = lax.iota(jnp.int32, 16)

        def fire(h, buf, sem):
            pltpu.async_copy(data_hbm.at[idxT_v.at[h]], rows_v.at[buf], sem)

        def drain(buf, sem):
            pltpu.make_async_copy(
                outT_hbm.at[0, :, pl.ds(b0, RB)], rows_v.at[buf], sem
            ).wait()

        def process(h, buf):
            def b_body(bb, c2):
                b = bb * UNROLL
                for u in range(UNROLL):
                    v0 = rows_v[buf, b + u, pl.ds(0, 16)]
                    v1 = rows_v[buf, b + u, pl.ds(16, 16)]
                    col = jnp.full((16,), 0, jnp.int32) + (b + u)
                    plsc.store_scatter(trans_v, [lanes, col], v0)
                    plsc.store_scatter(trans_v, [lanes + 16, col], v1)
                return c2

            lax.fori_loop(0, RB // UNROLL, b_body, 0)
            pltpu.sync_copy(
                trans_v.at[:, pl.ds(0, RB)], outT_hbm.at[h, :, pl.ds(b0, RB)]
            )

        fire(0, 0, gsem0)

        def pair_body(p, carry):
            h0 = 2 * p
            drain(0, gsem0)
            fire(h0 + 1, 1, gsem1)
            process(h0, 0)
            drain(1, gsem1)

            @pl.when(p < H // 2 - 1)
            def _():
                fire(h0 + 2, 0, gsem0)

            process(h0 + 1, 1)
            return carry

        lax.fori_loop(0, H // 2, pair_body, 0)

    return k(xT, data)


def kernel(x, data):
    outT = _sc_gather(x.T, data)
    return jnp.transpose(outT, (2, 0, 1))
